# baseline (speedup 1.0000x reference)
import functools

import jax
import jax.numpy as jnp
import numpy as np
from jax import lax
from jax.experimental import pallas as pl
from jax.experimental.pallas import tpu as pltpu
from jax.experimental.pallas import tpu_sc as plsc

N = 10000
E = 320000
F = 128
NCLASS = 40

NPAD = 10240
NW = 32
K = 128
ETOT = E + N
C = 84
EPAD = NW * C * K
ROWS_PER_TILE = NPAD // 16
NR = NPAD // 128
NG = C * K // 16


@functools.cache
def _sc_kernels():
    mesh = plsc.VectorSubcoreMesh(core_axis_name="c", subcore_axis_name="s")

    @functools.partial(
        pl.kernel,
        mesh=mesh,
        compiler_params=pltpu.CompilerParams(needs_layout_passes=False),
        out_type=[
            jax.ShapeDtypeStruct((2, NR, 128), jnp.float32),
            jax.ShapeDtypeStruct((2, NR, 128), jnp.float32),
        ],
        scratch_types=[
            pltpu.VMEM((C * K,), jnp.int32),
            pltpu.VMEM((C * K,), jnp.int32),
            pltpu.VMEM((NR, 128), jnp.float32),
            pltpu.VMEM((NR, 128), jnp.float32),
            pltpu.VMEM((1, NR), jnp.int32),
            pltpu.VMEM_SHARED((NR, 128), jnp.float32),
            pltpu.VMEM_SHARED((NR, 128), jnp.float32),
        ],
    )
    def deg_kernel(src_hbm, dst_hbm, zeros_hbm, iota_hbm, out_o, out_i,
                   sidx, didx, acc_o, acc_i, idrows, sh_o, sh_i):
        c = lax.axis_index("c")
        s = lax.axis_index("s")
        w = s * 2 + c
        pltpu.sync_copy(zeros_hbm, acc_o)
        pltpu.sync_copy(zeros_hbm, acc_i)
        pltpu.sync_copy(iota_hbm, idrows)
        pltpu.sync_copy(src_hbm.at[w], sidx)
        pltpu.sync_copy(dst_hbm.at[w], didx)

        @pl.when(s < 10)
        def _zero():
            pltpu.sync_copy(zeros_hbm.at[pl.ds(s * 8, 8)], sh_o.at[pl.ds(s * 8, 8)])
            pltpu.sync_copy(zeros_hbm.at[pl.ds(s * 8, 8)], sh_i.at[pl.ds(s * 8, 8)])

        plsc.subcore_barrier()

        ones = jnp.full((16,), 1.0, jnp.float32)

        def body(g, carry):
            i0 = g * 16
            si = sidx[pl.ds(i0, 16)]
            plsc.addupdate_scatter(acc_o, [si >> 7, si & 127], ones)
            di = didx[pl.ds(i0, 16)]
            plsc.addupdate_scatter(acc_i, [di >> 7, di & 127], ones)
            return carry

        lax.fori_loop(jnp.int32(0), jnp.int32(NG), body, jnp.int32(0))
        pltpu.sync_copy(acc_o, sh_o.at[idrows.at[jnp.int32(0)]], add=True)
        pltpu.sync_copy(acc_i, sh_i.at[idrows.at[jnp.int32(0)]], add=True)
        plsc.subcore_barrier()

        @pl.when(s < 10)
        def _out():
            pltpu.sync_copy(sh_o.at[pl.ds(s * 8, 8)], out_o.at[c, pl.ds(s * 8, 8)])
            pltpu.sync_copy(sh_i.at[pl.ds(s * 8, 8)], out_i.at[c, pl.ds(s * 8, 8)])

    @functools.partial(
        pl.kernel,
        mesh=mesh,
        out_type=jax.ShapeDtypeStruct((2, NPAD, F), jnp.float32),
        scratch_types=[
            pltpu.VMEM((C // 2, K), jnp.int32),
            pltpu.VMEM((C // 2, K), jnp.int32),
            pltpu.VMEM((K, F), jnp.float32),
            pltpu.VMEM((K, F), jnp.float32),
            pltpu.VMEM_SHARED((NPAD, F), jnp.float32),
            pltpu.SemaphoreType.DMA,
            pltpu.SemaphoreType.DMA,
        ],
    )
    def agg_kernel(h_hbm, src_hbm, dst_hbm, zeros_hbm, out_hbm,
                   sidx, didx, rows0, rows1, acc, sg0, sg1):
        c = lax.axis_index("c")
        s = lax.axis_index("s")
        w = s * 2 + c
        r0 = s * ROWS_PER_TILE
        pltpu.sync_copy(zeros_hbm, acc.at[pl.ds(r0, ROWS_PER_TILE)])
        plsc.subcore_barrier()

        half_c = C // 2
        pairs = half_c // 2
        plast = jnp.int32(pairs - 1)
        for half in range(2):
            hh = jnp.int32(half)
            pltpu.sync_copy(src_hbm.at[w, hh], sidx)
            pltpu.sync_copy(dst_hbm.at[w, hh], didx)
            pltpu.async_copy(h_hbm.at[sidx.at[jnp.int32(0)]], rows0, sg0)

            def body(p, carry):
                j0 = p * 2
                pltpu.make_async_copy(h_hbm.at[sidx.at[j0]], rows0, sg0).wait()
                pltpu.async_copy(h_hbm.at[sidx.at[j0 + 1]], rows1, sg1)
                pltpu.sync_copy(rows0, acc.at[didx.at[j0]], add=True)
                pltpu.make_async_copy(h_hbm.at[sidx.at[j0 + 1]], rows1, sg1).wait()

                @pl.when(p < plast)
                def _pref():
                    pltpu.async_copy(h_hbm.at[sidx.at[j0 + 2]], rows0, sg0)

                pltpu.sync_copy(rows1, acc.at[didx.at[j0 + 1]], add=True)
                return carry

            lax.fori_loop(jnp.int32(0), jnp.int32(pairs), body, jnp.int32(0))
        plsc.subcore_barrier()
        pltpu.sync_copy(acc.at[pl.ds(r0, ROWS_PER_TILE)],
                        out_hbm.at[c, pl.ds(r0, ROWS_PER_TILE)])

    return deg_kernel, agg_kernel


_BM = 1024
_GRID = NPAD // _BM
_DB = _BM // 128


def _inv_block(deg_ref):
    d = deg_ref[0] + deg_ref[1]
    return lax.rsqrt(jnp.maximum(d, 1.0))


def _scale_rows(mat, inv):
    m3 = mat.reshape(_DB, 128, F)
    return (m3 * inv[:, :, None]).reshape(_BM, F)


def _l1_body(x_ref, w_ref, dego_ref, out_ref):
    h = jnp.dot(x_ref[...], w_ref[...], preferred_element_type=jnp.float32)
    out_ref[...] = _scale_rows(h, _inv_block(dego_ref))


def _mid_body(aggp_ref, degi_ref, b_ref, w_ref, dego_ref, out_ref):
    agg = aggp_ref[0] + aggp_ref[1]
    h = jnp.maximum(_scale_rows(agg, _inv_block(degi_ref)) + b_ref[...], 0.0)
    hw = jnp.dot(h, w_ref[...], preferred_element_type=jnp.float32)
    out_ref[...] = _scale_rows(hw, _inv_block(dego_ref))


def _final_body(aggp_ref, degi_ref, b_ref, w_ref, b3_ref, out_ref):
    agg = aggp_ref[0] + aggp_ref[1]
    h = jnp.maximum(_scale_rows(agg, _inv_block(degi_ref)) + b_ref[...], 0.0)
    logits = jnp.dot(h, w_ref[...], preferred_element_type=jnp.float32) + b3_ref[...]
    m = jnp.max(logits, axis=1, keepdims=True)
    lse = jnp.log(jnp.sum(jnp.exp(logits - m), axis=1, keepdims=True))
    out_ref[...] = logits - m - lse


_I0 = np.int32(0)
_deg_spec = pl.BlockSpec((2, _DB, 128), lambda i: (_I0, i, _I0))
_mat_spec = pl.BlockSpec((_BM, F), lambda i: (i, _I0))
_w_spec = pl.BlockSpec((F, F), lambda i: (_I0, _I0))
_b_spec = pl.BlockSpec((1, F), lambda i: (_I0, _I0))
_aggp_spec = pl.BlockSpec((2, _BM, F), lambda i: (_I0, i, _I0))

_l1_call = pl.pallas_call(
    _l1_body,
    grid=(_GRID,),
    in_specs=[_mat_spec, _w_spec, _deg_spec],
    out_specs=_mat_spec,
    out_shape=jax.ShapeDtypeStruct((NPAD, F), jnp.float32),
)

_mid_call = pl.pallas_call(
    _mid_body,
    grid=(_GRID,),
    in_specs=[_aggp_spec, _deg_spec, _b_spec, _w_spec, _deg_spec],
    out_specs=_mat_spec,
    out_shape=jax.ShapeDtypeStruct((NPAD, F), jnp.float32),
)

_final_call = pl.pallas_call(
    _final_body,
    grid=(_GRID,),
    in_specs=[_aggp_spec, _deg_spec, _b_spec, _w_spec, _b_spec],
    out_specs=_mat_spec,
    out_shape=jax.ShapeDtypeStruct((NPAD, F), jnp.float32),
)


def kernel(x, edge_index, W1, b1, W2, b2, W3, b3):
    loop = jnp.arange(N, dtype=jnp.int32)
    src = jnp.concatenate([edge_index[0].astype(jnp.int32), loop])
    dst = jnp.concatenate([edge_index[1].astype(jnp.int32), loop])
    pad = jnp.full((EPAD - ETOT,), N, dtype=jnp.int32)
    srcp = jnp.concatenate([src, pad])
    dstp = jnp.concatenate([dst, pad])
    src3 = srcp.reshape(NW, 2, C // 2, K)
    dst3 = dstp.reshape(NW, 2, C // 2, K)
    src2 = srcp.reshape(NW, C * K)
    dst2 = dstp.reshape(NW, C * K)

    x_pad = jnp.zeros((NPAD, F), jnp.float32).at[:N].set(x.astype(jnp.float32))
    zerosR = jnp.zeros((NR, 128), jnp.float32)
    iotaR = jnp.arange(NR, dtype=jnp.int32).reshape(1, NR)
    zerosF = jnp.zeros((ROWS_PER_TILE, F), jnp.float32)

    deg_kernel, agg_kernel = _sc_kernels()
    dego_p, degi_p = deg_kernel(src2, dst2, zerosR, iotaR)

    h1 = _l1_call(x_pad, W1.astype(jnp.float32), dego_p)
    agg1 = agg_kernel(h1, src3, dst3, zerosF)

    b1r = b1.astype(jnp.float32).reshape(1, F)
    h2 = _mid_call(agg1, degi_p, b1r, W2.astype(jnp.float32), dego_p)
    agg2 = agg_kernel(h2, src3, dst3, zerosF)

    W3p = jnp.zeros((F, F), jnp.float32).at[:, :NCLASS].set(W3.astype(jnp.float32))
    b3p = jnp.full((1, F), -1e30, jnp.float32).at[0, :NCLASS].set(b3.astype(jnp.float32))
    b2r = b2.astype(jnp.float32).reshape(1, F)
    out = _final_call(agg2, degi_p, b2r, W3p, b3p)
    return out[:N, :NCLASS].astype(jnp.float64)


# speedup vs baseline: 3.0246x; 3.0246x over previous
import functools

import jax
import jax.numpy as jnp
import numpy as np
from jax import lax
from jax.experimental import pallas as pl
from jax.experimental.pallas import tpu as pltpu
from jax.experimental.pallas import tpu_sc as plsc

N = 10000
E = 320000
F = 128
NCLASS = 40

NPAD = 10240
NW = 32
K = 128
ETOT = E + N
C = 82
EPAD = NW * C * K
NDUM = NPAD - N
ROWS_PER_TILE = NPAD // 16
NR = NPAD // 128
NG = C * K // 16


@functools.cache
def _sc_kernels():
    mesh = plsc.VectorSubcoreMesh(core_axis_name="c", subcore_axis_name="s")

    @functools.partial(
        pl.kernel,
        mesh=mesh,
        compiler_params=pltpu.CompilerParams(needs_layout_passes=False),
        out_type=[
            jax.ShapeDtypeStruct((2, NR, 128), jnp.float32),
            jax.ShapeDtypeStruct((2, NR, 128), jnp.float32),
        ],
        scratch_types=[
            pltpu.VMEM((C * K,), jnp.int32),
            pltpu.VMEM((C * K,), jnp.int32),
            pltpu.VMEM((NR, 128), jnp.float32),
            pltpu.VMEM((NR, 128), jnp.float32),
            pltpu.VMEM((1, NR), jnp.int32),
            pltpu.VMEM_SHARED((NR, 128), jnp.float32),
            pltpu.VMEM_SHARED((NR, 128), jnp.float32),
        ],
    )
    def deg_kernel(src_hbm, dst_hbm, zeros_hbm, iota_hbm, out_o, out_i,
                   sidx, didx, acc_o, acc_i, idrows, sh_o, sh_i):
        c = lax.axis_index("c")
        s = lax.axis_index("s")
        w = s * 2 + c
        pltpu.sync_copy(zeros_hbm, acc_o)
        pltpu.sync_copy(zeros_hbm, acc_i)
        pltpu.sync_copy(iota_hbm, idrows)
        pltpu.sync_copy(src_hbm.at[w], sidx)
        pltpu.sync_copy(dst_hbm.at[w], didx)

        @pl.when(s < 10)
        def _zero():
            pltpu.sync_copy(zeros_hbm.at[pl.ds(s * 8, 8)], sh_o.at[pl.ds(s * 8, 8)])
            pltpu.sync_copy(zeros_hbm.at[pl.ds(s * 8, 8)], sh_i.at[pl.ds(s * 8, 8)])

        plsc.subcore_barrier()

        ones = jnp.full((16,), 1.0, jnp.float32)

        def body(g, carry):
            i0 = g * 16
            si = sidx[pl.ds(i0, 16)]
            plsc.addupdate_scatter(acc_o, [si >> 7, si & 127], ones)
            di = didx[pl.ds(i0, 16)]
            plsc.addupdate_scatter(acc_i, [di >> 7, di & 127], ones)
            return carry

        lax.fori_loop(jnp.int32(0), jnp.int32(NG), body, jnp.int32(0))
        pltpu.sync_copy(acc_o, sh_o.at[idrows.at[jnp.int32(0)]], add=True)
        pltpu.sync_copy(acc_i, sh_i.at[idrows.at[jnp.int32(0)]], add=True)
        plsc.subcore_barrier()

        @pl.when(s < 10)
        def _out():
            pltpu.sync_copy(sh_o.at[pl.ds(s * 8, 8)], out_o.at[c, pl.ds(s * 8, 8)])
            pltpu.sync_copy(sh_i.at[pl.ds(s * 8, 8)], out_i.at[c, pl.ds(s * 8, 8)])

    @functools.partial(
        pl.kernel,
        mesh=mesh,
        out_type=jax.ShapeDtypeStruct((2, NPAD, F), jnp.float32),
        scratch_types=[
            pltpu.VMEM((C, K), jnp.int32),
            pltpu.VMEM((C, K), jnp.int32),
            pltpu.VMEM((K, F), jnp.float32),
            pltpu.VMEM_SHARED((NPAD, F), jnp.float32),
            pltpu.SemaphoreType.DMA,
        ],
    )
    def agg_kernel(h_hbm, src_hbm, dst_hbm, zeros_hbm, out_hbm,
                   sidx, didx, rows_v, acc, sem):
        c = lax.axis_index("c")
        s = lax.axis_index("s")
        w = s * 2 + c
        r0 = s * ROWS_PER_TILE
        pltpu.sync_copy(zeros_hbm, acc.at[pl.ds(r0, ROWS_PER_TILE)])
        pltpu.sync_copy(src_hbm.at[w], sidx)
        pltpu.sync_copy(dst_hbm.at[w], didx)
        plsc.subcore_barrier()

        def body(j, carry):
            pltpu.async_copy(h_hbm.at[sidx.at[j]], rows_v, sem).wait()
            pltpu.sync_copy(rows_v, acc.at[didx.at[j]], add=True)
            return carry

        lax.fori_loop(jnp.int32(0), jnp.int32(C), body, jnp.int32(0))
        plsc.subcore_barrier()
        pltpu.sync_copy(acc.at[pl.ds(r0, ROWS_PER_TILE)],
                        out_hbm.at[c, pl.ds(r0, ROWS_PER_TILE)])

    return deg_kernel, agg_kernel


_BM = 1024
_GRID = NPAD // _BM
_DB = _BM // 128


def _inv_block(deg_ref):
    d = deg_ref[0] + deg_ref[1]
    return lax.rsqrt(jnp.maximum(d, 1.0))


def _scale_rows(mat, inv):
    m3 = mat.reshape(_DB, 128, F)
    return (m3 * inv[:, :, None]).reshape(_BM, F)


def _l1_body(x_ref, w_ref, dego_ref, out_ref):
    h = jnp.dot(x_ref[...], w_ref[...], preferred_element_type=jnp.float32)
    out_ref[...] = _scale_rows(h, _inv_block(dego_ref))


def _mid_body(aggp_ref, degi_ref, b_ref, w_ref, dego_ref, out_ref):
    agg = aggp_ref[0] + aggp_ref[1]
    h = jnp.maximum(_scale_rows(agg, _inv_block(degi_ref)) + b_ref[...], 0.0)
    hw = jnp.dot(h, w_ref[...], preferred_element_type=jnp.float32)
    out_ref[...] = _scale_rows(hw, _inv_block(dego_ref))


def _final_body(aggp_ref, degi_ref, b_ref, w_ref, b3_ref, out_ref):
    agg = aggp_ref[0] + aggp_ref[1]
    h = jnp.maximum(_scale_rows(agg, _inv_block(degi_ref)) + b_ref[...], 0.0)
    logits = jnp.dot(h, w_ref[...], preferred_element_type=jnp.float32) + b3_ref[...]
    m = jnp.max(logits, axis=1, keepdims=True)
    lse = jnp.log(jnp.sum(jnp.exp(logits - m), axis=1, keepdims=True))
    out_ref[...] = logits - m - lse


_I0 = np.int32(0)
_deg_spec = pl.BlockSpec((2, _DB, 128), lambda i: (_I0, i, _I0))
_mat_spec = pl.BlockSpec((_BM, F), lambda i: (i, _I0))
_w_spec = pl.BlockSpec((F, F), lambda i: (_I0, _I0))
_b_spec = pl.BlockSpec((1, F), lambda i: (_I0, _I0))
_aggp_spec = pl.BlockSpec((2, _BM, F), lambda i: (_I0, i, _I0))

_l1_call = pl.pallas_call(
    _l1_body,
    grid=(_GRID,),
    in_specs=[_mat_spec, _w_spec, _deg_spec],
    out_specs=_mat_spec,
    out_shape=jax.ShapeDtypeStruct((NPAD, F), jnp.float32),
)

_mid_call = pl.pallas_call(
    _mid_body,
    grid=(_GRID,),
    in_specs=[_aggp_spec, _deg_spec, _b_spec, _w_spec, _deg_spec],
    out_specs=_mat_spec,
    out_shape=jax.ShapeDtypeStruct((NPAD, F), jnp.float32),
)

_final_call = pl.pallas_call(
    _final_body,
    grid=(_GRID,),
    in_specs=[_aggp_spec, _deg_spec, _b_spec, _w_spec, _b_spec],
    out_specs=_mat_spec,
    out_shape=jax.ShapeDtypeStruct((NPAD, F), jnp.float32),
)


def kernel(x, edge_index, W1, b1, W2, b2, W3, b3):
    loop = jnp.arange(N, dtype=jnp.int32)
    src = jnp.concatenate([edge_index[0].astype(jnp.int32), loop])
    dst = jnp.concatenate([edge_index[1].astype(jnp.int32), loop])
    pad = N + jnp.arange(EPAD - ETOT, dtype=jnp.int32) % NDUM
    srcp = jnp.concatenate([src, pad])
    dstp = jnp.concatenate([dst, pad])
    src3 = srcp.reshape(NW, C, K)
    dst3 = dstp.reshape(NW, C, K)
    src2 = srcp.reshape(NW, C * K)
    dst2 = dstp.reshape(NW, C * K)

    x_pad = jnp.zeros((NPAD, F), jnp.float32).at[:N].set(x.astype(jnp.float32))
    zerosR = jnp.zeros((NR, 128), jnp.float32)
    iotaR = jnp.arange(NR, dtype=jnp.int32).reshape(1, NR)
    zerosF = jnp.zeros((ROWS_PER_TILE, F), jnp.float32)

    deg_kernel, agg_kernel = _sc_kernels()
    dego_p, degi_p = deg_kernel(src2, dst2, zerosR, iotaR)

    h1 = _l1_call(x_pad, W1.astype(jnp.float32), dego_p)
    agg1 = agg_kernel(h1, src3, dst3, zerosF)

    b1r = b1.astype(jnp.float32).reshape(1, F)
    h2 = _mid_call(agg1, degi_p, b1r, W2.astype(jnp.float32), dego_p)
    agg2 = agg_kernel(h2, src3, dst3, zerosF)

    W3p = jnp.zeros((F, F), jnp.float32).at[:, :NCLASS].set(W3.astype(jnp.float32))
    b3p = jnp.full((1, F), -1e30, jnp.float32).at[0, :NCLASS].set(b3.astype(jnp.float32))
    b2r = b2.astype(jnp.float32).reshape(1, F)
    out = _final_call(agg2, degi_p, b2r, W3p, b3p)
    return out[:N, :NCLASS].astype(jnp.float64)


# speedup vs baseline: 3.5950x; 1.1886x over previous
import functools

import jax
import jax.numpy as jnp
import numpy as np
from jax import lax
from jax.experimental import pallas as pl
from jax.experimental.pallas import tpu as pltpu
from jax.experimental.pallas import tpu_sc as plsc

N = 10000
E = 320000
F = 128
NCLASS = 40

NPAD = 10240
NW = 32
K = 128
ETOT = E + N
C = 84
EPAD = NW * C * K
NDUM = NPAD - N
ROWS_PER_TILE = NPAD // 16
NR = NPAD // 128
NG = C * K // 16


@functools.cache
def _sc_kernels():
    mesh = plsc.VectorSubcoreMesh(core_axis_name="c", subcore_axis_name="s")

    @functools.partial(
        pl.kernel,
        mesh=mesh,
        compiler_params=pltpu.CompilerParams(needs_layout_passes=False),
        out_type=[
            jax.ShapeDtypeStruct((2, NR, 128), jnp.float32),
            jax.ShapeDtypeStruct((2, NR, 128), jnp.float32),
        ],
        scratch_types=[
            pltpu.VMEM((C * K,), jnp.int32),
            pltpu.VMEM((C * K,), jnp.int32),
            pltpu.VMEM((NR, 128), jnp.float32),
            pltpu.VMEM((NR, 128), jnp.float32),
            pltpu.VMEM((1, NR), jnp.int32),
            pltpu.VMEM_SHARED((NR, 128), jnp.float32),
            pltpu.VMEM_SHARED((NR, 128), jnp.float32),
        ],
    )
    def deg_kernel(src_hbm, dst_hbm, zeros_hbm, iota_hbm, out_o, out_i,
                   sidx, didx, acc_o, acc_i, idrows, sh_o, sh_i):
        c = lax.axis_index("c")
        s = lax.axis_index("s")
        w = s * 2 + c
        pltpu.sync_copy(zeros_hbm, acc_o)
        pltpu.sync_copy(zeros_hbm, acc_i)
        pltpu.sync_copy(iota_hbm, idrows)
        pltpu.sync_copy(src_hbm.at[w], sidx)
        pltpu.sync_copy(dst_hbm.at[w], didx)

        @pl.when(s < 10)
        def _zero():
            pltpu.sync_copy(zeros_hbm.at[pl.ds(s * 8, 8)], sh_o.at[pl.ds(s * 8, 8)])
            pltpu.sync_copy(zeros_hbm.at[pl.ds(s * 8, 8)], sh_i.at[pl.ds(s * 8, 8)])

        plsc.subcore_barrier()

        ones = jnp.full((16,), 1.0, jnp.float32)

        def body(g, carry):
            i0 = g * 16
            si = sidx[pl.ds(i0, 16)]
            plsc.addupdate_scatter(acc_o, [si >> 7, si & 127], ones)
            di = didx[pl.ds(i0, 16)]
            plsc.addupdate_scatter(acc_i, [di >> 7, di & 127], ones)
            return carry

        lax.fori_loop(jnp.int32(0), jnp.int32(NG), body, jnp.int32(0))
        pltpu.sync_copy(acc_o, sh_o.at[idrows.at[jnp.int32(0)]], add=True)
        pltpu.sync_copy(acc_i, sh_i.at[idrows.at[jnp.int32(0)]], add=True)
        plsc.subcore_barrier()

        @pl.when(s < 10)
        def _out():
            pltpu.sync_copy(sh_o.at[pl.ds(s * 8, 8)], out_o.at[c, pl.ds(s * 8, 8)])
            pltpu.sync_copy(sh_i.at[pl.ds(s * 8, 8)], out_i.at[c, pl.ds(s * 8, 8)])

    @functools.partial(
        pl.kernel,
        mesh=mesh,
        out_type=jax.ShapeDtypeStruct((2, NPAD, F), jnp.float32),
        scratch_types=[
            pltpu.VMEM((C // 2, K), jnp.int32),
            pltpu.VMEM((C // 2, K), jnp.int32),
            pltpu.VMEM((K, F), jnp.float32),
            pltpu.VMEM((K, F), jnp.float32),
            pltpu.VMEM_SHARED((NPAD, F), jnp.float32),
            pltpu.SemaphoreType.DMA,
            pltpu.SemaphoreType.DMA,
        ],
    )
    def agg_kernel(h_hbm, src_hbm, dst_hbm, zeros_hbm, out_hbm,
                   sidx, didx, rows0, rows1, acc, ss0, ss1):
        c = lax.axis_index("c")
        s = lax.axis_index("s")
        w = s * 2 + c
        r0 = s * ROWS_PER_TILE
        pltpu.sync_copy(zeros_hbm, acc.at[pl.ds(r0, ROWS_PER_TILE)])
        plsc.subcore_barrier()

        half_c = C // 2
        pairs = half_c // 2
        k0 = pl.ds(jnp.int32(0), K)

        def drain(buf, sem):
            pltpu.make_async_copy(h_hbm.at[k0], buf, sem).wait()

        for half in range(2):
            hh = jnp.int32(half)
            pltpu.sync_copy(src_hbm.at[w, hh], sidx)
            pltpu.sync_copy(dst_hbm.at[w, hh], didx)

            def body(p, carry, first_half=(half == 0)):
                j0 = p * 2
                pltpu.sync_copy(h_hbm.at[sidx.at[j0]], rows0)
                if first_half:
                    @pl.when(p > 0)
                    def _w1():
                        drain(rows1, ss1)
                else:
                    drain(rows1, ss1)
                pltpu.async_copy(rows0, acc.at[didx.at[j0]], ss0, add=True)
                pltpu.sync_copy(h_hbm.at[sidx.at[j0 + 1]], rows1)
                drain(rows0, ss0)
                pltpu.async_copy(rows1, acc.at[didx.at[j0 + 1]], ss1, add=True)
                return carry

            lax.fori_loop(jnp.int32(0), jnp.int32(pairs), body, jnp.int32(0))
        drain(rows1, ss1)
        plsc.subcore_barrier()
        pltpu.sync_copy(acc.at[pl.ds(r0, ROWS_PER_TILE)],
                        out_hbm.at[c, pl.ds(r0, ROWS_PER_TILE)])

    return deg_kernel, agg_kernel


_BM = 1024
_GRID = NPAD // _BM
_DB = _BM // 128


def _inv_block(deg_ref):
    d = deg_ref[0] + deg_ref[1]
    return lax.rsqrt(jnp.maximum(d, 1.0))


def _scale_rows(mat, inv):
    m3 = mat.reshape(_DB, 128, F)
    return (m3 * inv[:, :, None]).reshape(_BM, F)


def _l1_body(x_ref, w_ref, dego_ref, out_ref):
    h = jnp.dot(x_ref[...], w_ref[...], preferred_element_type=jnp.float32)
    out_ref[...] = _scale_rows(h, _inv_block(dego_ref))


def _mid_body(aggp_ref, degi_ref, b_ref, w_ref, dego_ref, out_ref):
    agg = aggp_ref[0] + aggp_ref[1]
    h = jnp.maximum(_scale_rows(agg, _inv_block(degi_ref)) + b_ref[...], 0.0)
    hw = jnp.dot(h, w_ref[...], preferred_element_type=jnp.float32)
    out_ref[...] = _scale_rows(hw, _inv_block(dego_ref))


def _final_body(aggp_ref, degi_ref, b_ref, w_ref, b3_ref, out_ref):
    agg = aggp_ref[0] + aggp_ref[1]
    h = jnp.maximum(_scale_rows(agg, _inv_block(degi_ref)) + b_ref[...], 0.0)
    logits = jnp.dot(h, w_ref[...], preferred_element_type=jnp.float32) + b3_ref[...]
    m = jnp.max(logits, axis=1, keepdims=True)
    lse = jnp.log(jnp.sum(jnp.exp(logits - m), axis=1, keepdims=True))
    out_ref[...] = logits - m - lse


_I0 = np.int32(0)
_deg_spec = pl.BlockSpec((2, _DB, 128), lambda i: (_I0, i, _I0))
_mat_spec = pl.BlockSpec((_BM, F), lambda i: (i, _I0))
_w_spec = pl.BlockSpec((F, F), lambda i: (_I0, _I0))
_b_spec = pl.BlockSpec((1, F), lambda i: (_I0, _I0))
_aggp_spec = pl.BlockSpec((2, _BM, F), lambda i: (_I0, i, _I0))

_l1_call = pl.pallas_call(
    _l1_body,
    grid=(_GRID,),
    in_specs=[_mat_spec, _w_spec, _deg_spec],
    out_specs=_mat_spec,
    out_shape=jax.ShapeDtypeStruct((NPAD, F), jnp.float32),
)

_mid_call = pl.pallas_call(
    _mid_body,
    grid=(_GRID,),
    in_specs=[_aggp_spec, _deg_spec, _b_spec, _w_spec, _deg_spec],
    out_specs=_mat_spec,
    out_shape=jax.ShapeDtypeStruct((NPAD, F), jnp.float32),
)

_final_call = pl.pallas_call(
    _final_body,
    grid=(_GRID,),
    in_specs=[_aggp_spec, _deg_spec, _b_spec, _w_spec, _b_spec],
    out_specs=_mat_spec,
    out_shape=jax.ShapeDtypeStruct((NPAD, F), jnp.float32),
)


def kernel(x, edge_index, W1, b1, W2, b2, W3, b3):
    loop = jnp.arange(N, dtype=jnp.int32)
    src = jnp.concatenate([edge_index[0].astype(jnp.int32), loop])
    dst = jnp.concatenate([edge_index[1].astype(jnp.int32), loop])
    pad = N + jnp.arange(EPAD - ETOT, dtype=jnp.int32) % NDUM
    srcp = jnp.concatenate([src, pad])
    dstp = jnp.concatenate([dst, pad])
    src3 = srcp.reshape(NW, 2, C // 2, K)
    dst3 = dstp.reshape(NW, 2, C // 2, K)
    src2 = srcp.reshape(NW, C * K)
    dst2 = dstp.reshape(NW, C * K)

    x_pad = jnp.zeros((NPAD, F), jnp.float32).at[:N].set(x.astype(jnp.float32))
    zerosR = jnp.zeros((NR, 128), jnp.float32)
    iotaR = jnp.arange(NR, dtype=jnp.int32).reshape(1, NR)
    zerosF = jnp.zeros((ROWS_PER_TILE, F), jnp.float32)

    deg_kernel, agg_kernel = _sc_kernels()
    dego_p, degi_p = deg_kernel(src2, dst2, zerosR, iotaR)

    h1 = _l1_call(x_pad, W1.astype(jnp.float32), dego_p)
    agg1 = agg_kernel(h1, src3, dst3, zerosF)

    b1r = b1.astype(jnp.float32).reshape(1, F)
    h2 = _mid_call(agg1, degi_p, b1r, W2.astype(jnp.float32), dego_p)
    agg2 = agg_kernel(h2, src3, dst3, zerosF)

    W3p = jnp.zeros((F, F), jnp.float32).at[:, :NCLASS].set(W3.astype(jnp.float32))
    b3p = jnp.full((1, F), -1e30, jnp.float32).at[0, :NCLASS].set(b3.astype(jnp.float32))
    b2r = b2.astype(jnp.float32).reshape(1, F)
    out = _final_call(agg2, degi_p, b2r, W3p, b3p)
    return out[:N, :NCLASS].astype(jnp.float64)
